# baseline (speedup 1.0000x reference)
import functools
import jax
import jax.numpy as jnp
import numpy as np
from jax import lax
from jax.experimental import pallas as pl
from jax.experimental.pallas import tpu as pltpu
from jax.experimental.pallas import tpu_sc as plsc

N_ITERS = 4
_NW = 32


def _sc_gather_multi(specs):
    n = len(specs)
    budget = 98304 // n
    plans = []
    for table, idx in specs:
        V, D = table.shape
        (M,) = idx.shape
        assert D % 128 == 0 and M % (8 * _NW) == 0, (V, D, M)
        b_per_w = M // _NW
        ch = b_per_w
        while ch * D > budget or ch > 2048:
            ch //= 2
        if ch % 8 != 0 or b_per_w % ch != 0:
            ch = 8
        assert ch >= 8 and b_per_w % ch == 0, (b_per_w, ch, D)
        plans.append((V, D, M, b_per_w, ch))

    scratch = []
    for (V, D, M, b_per_w, ch) in plans:
        scratch.append(pltpu.VMEM((ch,), jnp.int32))
        scratch.append(pltpu.VMEM((ch, D), jnp.float32))
    scratch.append(pltpu.SemaphoreType.DMA)

    mesh = plsc.VectorSubcoreMesh(core_axis_name="c", subcore_axis_name="s")

    @functools.partial(
        pl.kernel, mesh=mesh,
        out_type=[jax.ShapeDtypeStruct((M, D), jnp.float32)
                  for (V, D, M, b, c) in plans],
        scratch_types=scratch,
    )
    def k(*refs):
        ins = refs[:2 * n]
        outs = refs[2 * n:3 * n]
        scr = refs[3 * n:]
        sem = scr[-1]
        wid = lax.axis_index("s") * 2 + lax.axis_index("c")
        for t in range(n):
            V, D, M, b_per_w, ch = plans[t]
            table_hbm = ins[2 * t]
            idx_hbm = ins[2 * t + 1]
            out_hbm = outs[t]
            idx_v = scr[2 * t]
            rows_v = scr[2 * t + 1]
            base = wid * b_per_w

            def body(i, _, table_hbm=table_hbm, idx_hbm=idx_hbm,
                     out_hbm=out_hbm, idx_v=idx_v, rows_v=rows_v,
                     base=base, ch=ch):
                off = base + i * ch
                pltpu.sync_copy(idx_hbm.at[pl.ds(off, ch)], idx_v)
                pltpu.async_copy(table_hbm.at[idx_v], rows_v, sem).wait()
                pltpu.sync_copy(rows_v, out_hbm.at[pl.ds(off, ch)])
                return 0

            lax.fori_loop(0, b_per_w // ch, body, 0, unroll=False)

    flat_in = []
    for table, idx in specs:
        flat_in += [table, idx]
    outs = k(*flat_in)
    return list(outs) if isinstance(outs, (list, tuple)) else [outs]


def _flat_idx(idx, V):
    B, Nq, k = idx.shape
    off = (jnp.arange(B, dtype=jnp.int32) * V)[:, None, None]
    return jnp.transpose(idx + off, (2, 0, 1)).reshape(-1)


def _pad128(t):
    D = t.shape[-1]
    Dp = ((D + 127) // 128) * 128
    if Dp == D:
        return t
    return jnp.pad(t, ((0, 0), (0, Dp - D)))


def _knn_body(k, Ni, P, q_ref, i_ref, o_ref):
    q = q_ref[0]
    ix = i_ref[0]
    qd = jnp.dot(q, ix, preferred_element_type=jnp.float32)
    qn = q[:, 0:1] ** 2 + q[:, 1:2] ** 2 + q[:, 2:3] ** 2
    xn = ix[0:1, :] ** 2 + ix[1:2, :] ** 2 + ix[2:3, :] ** 2
    d = qn + xn - 2.0 * qd
    iota = lax.broadcasted_iota(jnp.int32, (P, Ni), 1)
    cols = []
    for _ in range(k):
        m = jnp.min(d, axis=1, keepdims=True)
        cand = jnp.where(d == m, iota, Ni)
        amin = jnp.min(cand, axis=1, keepdims=True)
        cols.append(amin)
        d = jnp.where(cand == amin, jnp.inf, d)
    o_ref[0] = jnp.concatenate(cols, axis=1)


def _knn(input_xyz, query_n, k, P=256):
    B, _, Ni = input_xyz.shape
    Nq = query_n.shape[1]
    return pl.pallas_call(
        functools.partial(_knn_body, k, Ni, P),
        grid=(B, Nq // P),
        in_specs=[
            pl.BlockSpec((1, P, 3), lambda b, t: (b, t, 0)),
            pl.BlockSpec((1, 3, Ni), lambda b, t: (b, 0, 0)),
        ],
        out_specs=pl.BlockSpec((1, P, k), lambda b, t: (b, t, 0)),
        out_shape=jax.ShapeDtypeStruct((B, Nq, k), jnp.int32),
    )(query_n, input_xyz)


def _act(h, a):
    if a == 'lrelu':
        return jax.nn.leaky_relu(h, 0.1)
    if a == 'relu':
        return jax.nn.relu(h)
    return h


def _pw_body(acts, *refs):
    x_ref = refs[0]
    o_ref = refs[-1]
    h = x_ref[0]
    for i, a in enumerate(acts):
        w_ref = refs[1 + 2 * i]
        b_ref = refs[2 + 2 * i]
        h = jnp.dot(h, w_ref[...], preferred_element_type=jnp.float32) \
            + b_ref[...]
        h = _act(h, a)
    o_ref[0] = h


def _pw(x, stages, P=256):
    G, N, Cin = x.shape
    acts = tuple(a for (_, a) in stages)
    ins = [x]
    in_specs = [pl.BlockSpec((1, P, Cin), lambda g, t: (g, t, 0))]
    for (p, a) in stages:
        co, ci = p['w'].shape
        ins.append(jnp.transpose(p['w']))
        ins.append(p['b'].reshape(1, co))
        in_specs.append(pl.BlockSpec((ci, co), lambda g, t: (0, 0)))
        in_specs.append(pl.BlockSpec((1, co), lambda g, t: (0, 0)))
    Cout = stages[-1][0]['w'].shape[0]
    return pl.pallas_call(
        functools.partial(_pw_body, acts),
        grid=(G, N // P),
        in_specs=in_specs,
        out_specs=pl.BlockSpec((1, P, Cout), lambda g, t: (g, t, 0)),
        out_shape=jax.ShapeDtypeStruct((G, N, Cout), jnp.float32),
    )(*ins)


def _bmm_body(x_ref, y_ref, o_ref):
    o_ref[0] = jnp.dot(x_ref[0], y_ref[0],
                       preferred_element_type=jnp.float32)


def _bmm(x, y, P=256):
    B, M, C = x.shape
    N = y.shape[2]
    return pl.pallas_call(
        _bmm_body,
        grid=(B, M // P),
        in_specs=[
            pl.BlockSpec((1, P, C), lambda b, t: (b, t, 0)),
            pl.BlockSpec((1, C, N), lambda b, t: (b, 0, 0)),
        ],
        out_specs=pl.BlockSpec((1, P, N), lambda b, t: (b, t, 0)),
        out_shape=jax.ShapeDtypeStruct((B, M, N), jnp.float32),
    )(x, y)


def _lrelu(x):
    return jax.nn.leaky_relu(x, 0.1)


def _conv1d_n(p, x):
    return _pw(x, [(p, 'none')])


def _build_pc_pyramid(pc, n_samples):
    xyzs = [pc]
    cur = pc
    for n in n_samples:
        cur = cur[:, :, :n]
        xyzs.append(cur)
    return xyzs


def _point_conv_post(p, g, xyz_qn, C):
    k, B, Nq, _ = g.shape
    rel = g[..., :3] - xyz_qn[None]
    cat = jnp.concatenate([rel, g[..., 3:3 + C]], axis=-1)
    h = _pw(cat.reshape(k * B, Nq, 3 + C), [(p, 'none')])
    h = h.reshape(k, B, Nq, -1)
    h = _lrelu(h * p['gamma'] + p['beta'])
    return jnp.max(h, axis=0)


def _dw_post(nf, w, C, act=True):
    out = jnp.mean(w * nf[..., :C], axis=0)
    return _lrelu(out) if act else out


def _interp_post(g, query_n):
    nxyz = g[..., :3]
    nfeat = g[..., 3:6]
    d = jnp.sqrt(jnp.sum((nxyz - query_n[None]) ** 2, axis=-1) + 1e-12)
    w = 1.0 / (d + 1e-8)
    w = w / jnp.sum(w, axis=0, keepdims=True)
    return jnp.sum(nfeat * w[..., None], axis=0)


def kernel(pc1, pc2, params):
    B = pc1.shape[0]
    xyzs1 = _build_pc_pyramid(pc1, [4096, 2048, 1024, 512, 256])
    xyzs2 = _build_pc_pyramid(pc2, [4096, 2048, 1024, 512, 256])
    xyzs1n = [jnp.transpose(x, (0, 2, 1)) for x in xyzs1]
    xyzs2n = [jnp.transpose(x, (0, 2, 1)) for x in xyzs2]

    encs = [(params['fnet'], xyzs1, xyzs1n), (params['fnet'], xyzs2, xyzs2n),
            (params['cnet'], xyzs1, xyzs1n)]
    f_lvl = [_pw(xn[0], [(p['level0'][0], 'lrelu'), (p['level0'][1], 'lrelu'),
                         (p['mlps'][0][0], 'lrelu'), (p['mlps'][0][1], 'lrelu')])
             for (p, xz, xn) in encs]

    specs = []
    for (p, xz, xn), f in zip(encs, f_lvl):
        idx = _knn(xz[0], xn[1], 16)
        table = _pad128(jnp.concatenate(
            [xn[0], f], axis=-1).reshape(B * xz[0].shape[2], -1))
        specs.append((table, _flat_idx(idx, xz[0].shape[2])))
    gs = _sc_gather_multi(specs)

    feats1 = []
    for (p, xz, xn), g in zip(encs, gs):
        Nq = xz[1].shape[2]
        gg = g.reshape(16, B, Nq, -1)
        feats1.append(_point_conv_post(p['convs'][0], gg, xn[1], 96))

    f_lvl2 = [_pw(f, [(e[0]['mlps'][1][0], 'lrelu'),
                      (e[0]['mlps'][1][1], 'lrelu')])
              for e, f in zip(encs, feats1)]

    xyz1 = xyzs1[2]
    xyz1n = xyzs1n[2]
    specs = []
    for (p, xz, xn), f in zip(encs, f_lvl2):
        idx = _knn(xz[1], xn[2], 16)
        table = _pad128(jnp.concatenate(
            [xn[1], f], axis=-1).reshape(B * xz[1].shape[2], -1))
        specs.append((table, _flat_idx(idx, xz[1].shape[2])))
    knn_idx = _knn(xyz1, xyz1n, 32)
    xyz1_rows = _pad128(xyz1n.reshape(B * 2048, 3))
    specs.append((xyz1_rows, _flat_idx(knn_idx, 2048)))
    gs = _sc_gather_multi(specs)

    feats2 = []
    for (p, xz, xn), g in zip(encs, gs[:3]):
        gg = g.reshape(16, B, 2048, -1)
        feats2.append(_point_conv_post(p['convs'][1], gg, xn[2], 128))
    feat1, feat2, featc = feats2
    featc = _conv1d_n(params['cnet_aligner'], featc)

    rel32 = gs[3].reshape(32, B, 2048, -1)[..., :3] - xyz1n[None]

    mp = params['motion_encoder']
    gp = params['gru']
    fp = params['flow_head']
    cp = params['correlation']

    wnets = [mp['conv_c1'], mp['conv_f1'], mp['conv_f2'], mp['conv'],
             gp['conv_z'], gp['conv_r'], gp['conv_q'],
             fp['conv1'], fp['conv2']]
    wcat = {'w': jnp.concatenate([q['wnet']['w'] for q in wnets], axis=0),
            'b': jnp.concatenate([q['wnet']['b'] for q in wnets], axis=0)}
    wall = _pw(rel32.reshape(32 * B, 2048, 3), [(wcat, 'none')])
    wall = wall.reshape(32, B, 2048, -1)
    wf = {}
    off = 0
    for name, q in zip(['c1', 'f1', 'f2', 'mc', 'z', 'r', 'q', 'fh1', 'fh2'],
                       wnets):
        C = q['wnet']['w'].shape[0]
        wf[name] = wall[..., off:off + C]
        off += C

    cvT = [_bmm(feat2, jnp.transpose(feat1, (0, 2, 1))) / 128.0]
    Ns2 = [2048, 1024, 512, 256]
    for i in range(1, 4):
        idx = _knn(xyzs2[2 + i - 1], xyzs2n[2 + i], 3)
        table = cvT[i - 1].reshape(B * Ns2[i - 1], 2048)
        g = _sc_gather_multi([(table, _flat_idx(idx, Ns2[i - 1]))])[0]
        cvT.append(jnp.mean(g.reshape(3, B, Ns2[i], 2048), axis=0))
    cv_flat = [c.reshape(B * Ns2[i] * 16, 128) for i, c in enumerate(cvT)]

    h = jnp.tanh(featc[..., :128])
    x = jax.nn.relu(featc[..., 128:])

    xyzs2c = xyzs2[2:]
    xyzs2cn = xyzs2n[2:]
    lane_eye = jnp.eye(128, dtype=jnp.float32)
    n_idx = jnp.arange(2048, dtype=jnp.int32)

    flow_preds = []
    flow_pred = jnp.zeros_like(xyz1n)
    for it in range(N_ITERS):
        if it > 0:
            table = _pad128(jnp.concatenate(
                [xyz1n, flow_pred], axis=-1).reshape(B * 2048, 6))
            specs = []
            for ln in xyzs2cn:
                idx = _knn(xyz1, ln, 3)
                specs.append((table, _flat_idx(idx, 2048)))
            gs = _sc_gather_multi(specs)
            xyzs2_warp = []
            xyzs2_warp_n = []
            for ln, g in zip(xyzs2cn, gs):
                Nl = ln.shape[1]
                wn = ln - _interp_post(g.reshape(3, B, Nl, -1), ln)
                xyzs2_warp_n.append(wn)
                xyzs2_warp.append(jnp.transpose(wn, (0, 2, 1)))
        else:
            xyzs2_warp = xyzs2c
            xyzs2_warp_n = xyzs2cn

        specs = []
        idxs = []
        for i, xw in enumerate(xyzs2_warp):
            Ni = xw.shape[2]
            idx = _knn(xw, xyz1n, 16)
            idxs.append(idx)
            specs.append((_pad128(xyzs2_warp_n[i].reshape(B * Ni, 3)),
                          _flat_idx(idx, Ni)))
        for i in range(4):
            Ni = xyzs2_warp[i].shape[2]
            boff = (jnp.arange(B, dtype=jnp.int32) * Ni)[:, None, None]
            frow = (idxs[i] + boff) * 16 + (n_idx[None, :, None] // 128)
            specs.append((cv_flat[i],
                          jnp.transpose(frow, (2, 0, 1)).reshape(-1)))
        gs = _sc_gather_multi(specs)

        feats = []
        for i in range(4):
            kxyz = gs[i].reshape(16, B, 2048, -1)[..., :3]
            rel = kxyz - xyz1n[None]
            gflat = gs[4 + i].reshape(16, B, 16, 128, 128)
            corr = jnp.sum(gflat * lane_eye, axis=-1).reshape(16, B, 2048)
            feats.append(jnp.concatenate([rel, corr[..., None]], axis=-1))
        feats = jnp.stack(feats, axis=0)
        hc = _pw(feats.reshape(4 * 16 * B, 2048, 4),
                 [(cp['cost_mlp'][0], 'relu'), (cp['cost_mlp'][1], 'relu')])
        hc = jnp.sum(hc.reshape(4, 16, B, 2048, 32), axis=1)
        costs = jnp.concatenate([hc[i] for i in range(4)], axis=-1)
        corr = _pw(costs, [(cp['merge'], 'lrelu')])

        lin_c1 = _conv1d_n(mp['conv_c1']['lin'], corr)
        lin_f1 = _conv1d_n(mp['conv_f1']['lin'], flow_pred)
        gs = _sc_gather_multi([
            (lin_c1.reshape(B * 2048, 128),
             _flat_idx(knn_idx[:, :, :16], 2048)),
            (_pad128(lin_f1.reshape(B * 2048, 32)), _flat_idx(knn_idx, 2048)),
        ])
        corr_feat = _dw_post(gs[0].reshape(16, B, 2048, -1), wf['c1'][:16], 128)
        flow_feat = _dw_post(gs[1].reshape(32, B, 2048, -1), wf['f1'], 32)
        lin_f2 = _conv1d_n(mp['conv_f2']['lin'], flow_feat)
        g = _sc_gather_multi([(_pad128(lin_f2.reshape(B * 2048, 16)),
                               _flat_idx(knn_idx[:, :, :16], 2048))])[0]
        flow_feat = _dw_post(g.reshape(16, B, 2048, -1), wf['f2'][:16], 16)
        lin_mc = _conv1d_n(mp['conv']['lin'],
                           jnp.concatenate([corr_feat, flow_feat], axis=-1))
        g = _sc_gather_multi([(_pad128(lin_mc.reshape(B * 2048, 125)),
                               _flat_idx(knn_idx[:, :, :16], 2048))])[0]
        mfeat = _dw_post(g.reshape(16, B, 2048, -1), wf['mc'][:16], 125)
        motion_feat = jnp.concatenate([mfeat, flow_pred], axis=-1)

        xmf = jnp.concatenate([x, motion_feat], axis=-1)
        hx = jnp.concatenate([h, xmf], axis=-1)
        lin_z = _conv1d_n(gp['conv_z']['lin'], hx)
        lin_r = _conv1d_n(gp['conv_r']['lin'], hx)
        gs = _sc_gather_multi([
            (lin_z.reshape(B * 2048, 128), _flat_idx(knn_idx[:, :, :4], 2048)),
            (lin_r.reshape(B * 2048, 128), _flat_idx(knn_idx[:, :, :4], 2048)),
        ])
        z = jax.nn.sigmoid(_dw_post(gs[0].reshape(4, B, 2048, -1),
                                    wf['z'][:4], 128, act=False))
        r = jax.nn.sigmoid(_dw_post(gs[1].reshape(4, B, 2048, -1),
                                    wf['r'][:4], 128, act=False))
        lin_q = _conv1d_n(gp['conv_q']['lin'],
                          jnp.concatenate([r * h, xmf], axis=-1))
        g = _sc_gather_multi([(lin_q.reshape(B * 2048, 128),
                               _flat_idx(knn_idx[:, :, :4], 2048))])[0]
        q = jnp.tanh(_dw_post(g.reshape(4, B, 2048, -1),
                              wf['q'][:4], 128, act=False))
        h = (1 - z) * h + z * q

        lin1 = _conv1d_n(fp['conv1']['lin'], h)
        g = _sc_gather_multi([(lin1.reshape(B * 2048, 128),
                               _flat_idx(knn_idx, 2048))])[0]
        f = _dw_post(g.reshape(32, B, 2048, -1), wf['fh1'], 128)
        lin2 = _conv1d_n(fp['conv2']['lin'], f)
        g = _sc_gather_multi([(_pad128(lin2.reshape(B * 2048, 64)),
                               _flat_idx(knn_idx, 2048))])[0]
        f = _dw_post(g.reshape(32, B, 2048, -1), wf['fh2'], 64)
        flow_delta = _conv1d_n(fp['fc'], f)
        flow_pred = flow_pred + flow_delta
        flow_preds.append(flow_pred)

    idx_up = _knn(xyz1, xyzs1n[0], 3)
    fidx = _flat_idx(idx_up, 2048)
    specs = [(_pad128(jnp.concatenate([xyz1n, fpred], axis=-1)
                      .reshape(B * 2048, 6)), fidx)
             for fpred in flow_preds]
    gs = _sc_gather_multi(specs)
    ups = [jnp.transpose(_interp_post(g.reshape(3, B, 8192, -1), xyzs1n[0]),
                         (0, 2, 1)) for g in gs]
    return jnp.stack(ups, axis=0)


# speedup vs baseline: 1.2435x; 1.2435x over previous
import functools
import jax
import jax.numpy as jnp
import numpy as np
from jax import lax
from jax.experimental import pallas as pl
from jax.experimental.pallas import tpu as pltpu
from jax.experimental.pallas import tpu_sc as plsc

N_ITERS = 4
_NW = 32


def _sc_gather_multi(specs):
    n = len(specs)
    budget = 98304 // n
    plans = []
    for table, idx in specs:
        V, D = table.shape
        (M,) = idx.shape
        assert D % 128 == 0 and M % (8 * _NW) == 0, (V, D, M)
        b_per_w = M // _NW
        ch = b_per_w
        while ch * D > budget or ch > 2048:
            ch //= 2
        if ch % 8 != 0 or b_per_w % ch != 0:
            ch = 8
        assert ch >= 8 and b_per_w % ch == 0, (b_per_w, ch, D)
        plans.append((V, D, M, b_per_w, ch))

    scratch = []
    for (V, D, M, b_per_w, ch) in plans:
        scratch.append(pltpu.VMEM((ch,), jnp.int32))
        scratch.append(pltpu.VMEM((ch, D), jnp.float32))
    scratch.append(pltpu.SemaphoreType.DMA)

    mesh = plsc.VectorSubcoreMesh(core_axis_name="c", subcore_axis_name="s")

    @functools.partial(
        pl.kernel, mesh=mesh,
        out_type=[jax.ShapeDtypeStruct((M, D), jnp.float32)
                  for (V, D, M, b, c) in plans],
        scratch_types=scratch,
    )
    def k(*refs):
        ins = refs[:2 * n]
        outs = refs[2 * n:3 * n]
        scr = refs[3 * n:]
        sem = scr[-1]
        wid = lax.axis_index("s") * 2 + lax.axis_index("c")
        for t in range(n):
            V, D, M, b_per_w, ch = plans[t]
            table_hbm = ins[2 * t]
            idx_hbm = ins[2 * t + 1]
            out_hbm = outs[t]
            idx_v = scr[2 * t]
            rows_v = scr[2 * t + 1]
            base = wid * b_per_w

            def body(i, _, table_hbm=table_hbm, idx_hbm=idx_hbm,
                     out_hbm=out_hbm, idx_v=idx_v, rows_v=rows_v,
                     base=base, ch=ch):
                off = base + i * ch
                pltpu.sync_copy(idx_hbm.at[pl.ds(off, ch)], idx_v)
                pltpu.async_copy(table_hbm.at[idx_v], rows_v, sem).wait()
                pltpu.sync_copy(rows_v, out_hbm.at[pl.ds(off, ch)])
                return 0

            lax.fori_loop(0, b_per_w // ch, body, 0, unroll=False)

    flat_in = []
    for table, idx in specs:
        flat_in += [table, idx]
    outs = k(*flat_in)
    return list(outs) if isinstance(outs, (list, tuple)) else [outs]


def _flat_idx(idx, V):
    B, Nq, k = idx.shape
    off = (jnp.arange(B, dtype=jnp.int32) * V)[:, None, None]
    return jnp.transpose(idx + off, (2, 0, 1)).reshape(-1)


def _pad128(t):
    D = t.shape[-1]
    Dp = ((D + 127) // 128) * 128
    if Dp == D:
        return t
    return jnp.pad(t, ((0, 0), (0, Dp - D)))


def _knn_body(k, Ni, P, q_ref, i_ref, o_ref):
    q = q_ref[0]
    ix = i_ref[0]
    qd = jnp.dot(q, ix, preferred_element_type=jnp.float32)
    qn = q[:, 0:1] ** 2 + q[:, 1:2] ** 2 + q[:, 2:3] ** 2
    xn = ix[0:1, :] ** 2 + ix[1:2, :] ** 2 + ix[2:3, :] ** 2
    d = qn + xn - 2.0 * qd
    iota = lax.broadcasted_iota(jnp.int32, (P, Ni), 1)
    cols = []
    for _ in range(k):
        m = jnp.min(d, axis=1, keepdims=True)
        cand = jnp.where(d == m, iota, Ni)
        amin = jnp.min(cand, axis=1, keepdims=True)
        cols.append(amin)
        d = jnp.where(cand == amin, jnp.inf, d)
    o_ref[0] = jnp.concatenate(cols, axis=1)


def _knn(input_xyz, query_n, k, P=256):
    B, _, Ni = input_xyz.shape
    Nq = query_n.shape[1]
    return pl.pallas_call(
        functools.partial(_knn_body, k, Ni, P),
        grid=(B, Nq // P),
        in_specs=[
            pl.BlockSpec((1, P, 3), lambda b, t: (b, t, 0)),
            pl.BlockSpec((1, 3, Ni), lambda b, t: (b, 0, 0)),
        ],
        out_specs=pl.BlockSpec((1, P, k), lambda b, t: (b, t, 0)),
        out_shape=jax.ShapeDtypeStruct((B, Nq, k), jnp.int32),
    )(query_n, input_xyz)


def _act(h, a):
    if a == 'lrelu':
        return jax.nn.leaky_relu(h, 0.1)
    if a == 'relu':
        return jax.nn.relu(h)
    return h


def _pw_body(acts, *refs):
    x_ref = refs[0]
    o_ref = refs[-1]
    h = x_ref[0]
    for i, a in enumerate(acts):
        w_ref = refs[1 + 2 * i]
        b_ref = refs[2 + 2 * i]
        h = jnp.dot(h, w_ref[...], preferred_element_type=jnp.float32) \
            + b_ref[...]
        h = _act(h, a)
    o_ref[0] = h


def _pw(x, stages, P=256):
    G, N, Cin = x.shape
    acts = tuple(a for (_, a) in stages)
    ins = [x]
    in_specs = [pl.BlockSpec((1, P, Cin), lambda g, t: (g, t, 0))]
    for (p, a) in stages:
        co, ci = p['w'].shape
        ins.append(jnp.transpose(p['w']))
        ins.append(p['b'].reshape(1, co))
        in_specs.append(pl.BlockSpec((ci, co), lambda g, t: (0, 0)))
        in_specs.append(pl.BlockSpec((1, co), lambda g, t: (0, 0)))
    Cout = stages[-1][0]['w'].shape[0]
    return pl.pallas_call(
        functools.partial(_pw_body, acts),
        grid=(G, N // P),
        in_specs=in_specs,
        out_specs=pl.BlockSpec((1, P, Cout), lambda g, t: (g, t, 0)),
        out_shape=jax.ShapeDtypeStruct((G, N, Cout), jnp.float32),
    )(*ins)


def _bmm_body(x_ref, y_ref, o_ref):
    o_ref[0] = jnp.dot(x_ref[0], y_ref[0],
                       preferred_element_type=jnp.float32)


def _bmm(x, y, P=256):
    B, M, C = x.shape
    N = y.shape[2]
    return pl.pallas_call(
        _bmm_body,
        grid=(B, M // P),
        in_specs=[
            pl.BlockSpec((1, P, C), lambda b, t: (b, t, 0)),
            pl.BlockSpec((1, C, N), lambda b, t: (b, 0, 0)),
        ],
        out_specs=pl.BlockSpec((1, P, N), lambda b, t: (b, t, 0)),
        out_shape=jax.ShapeDtypeStruct((B, M, N), jnp.float32),
    )(x, y)


def _lrelu(x):
    return jax.nn.leaky_relu(x, 0.1)


def _conv1d_n(p, x):
    return jnp.einsum('bnc,oc->bno', x, p['w']) + p['b']


def _build_pc_pyramid(pc, n_samples):
    xyzs = [pc]
    cur = pc
    for n in n_samples:
        cur = cur[:, :, :n]
        xyzs.append(cur)
    return xyzs


def _point_conv_post(p, g, xyz_qn, C):
    k, B, Nq, _ = g.shape
    rel = g[..., :3] - xyz_qn[None]
    cat = jnp.concatenate([rel, g[..., 3:3 + C]], axis=-1)
    h = _pw(cat.reshape(k * B, Nq, 3 + C), [(p, 'none')])
    h = h.reshape(k, B, Nq, -1)
    h = _lrelu(h * p['gamma'] + p['beta'])
    return jnp.max(h, axis=0)


def _dw_post(nf, w, C, act=True):
    out = jnp.mean(w * nf[..., :C], axis=0)
    return _lrelu(out) if act else out


def _interp_post(g, query_n):
    nxyz = g[..., :3]
    nfeat = g[..., 3:6]
    d = jnp.sqrt(jnp.sum((nxyz - query_n[None]) ** 2, axis=-1) + 1e-12)
    w = 1.0 / (d + 1e-8)
    w = w / jnp.sum(w, axis=0, keepdims=True)
    return jnp.sum(nfeat * w[..., None], axis=0)


def kernel(pc1, pc2, params):
    B = pc1.shape[0]
    xyzs1 = _build_pc_pyramid(pc1, [4096, 2048, 1024, 512, 256])
    xyzs2 = _build_pc_pyramid(pc2, [4096, 2048, 1024, 512, 256])
    xyzs1n = [jnp.transpose(x, (0, 2, 1)) for x in xyzs1]
    xyzs2n = [jnp.transpose(x, (0, 2, 1)) for x in xyzs2]

    encs = [(params['fnet'], xyzs1, xyzs1n), (params['fnet'], xyzs2, xyzs2n),
            (params['cnet'], xyzs1, xyzs1n)]
    f_lvl = [_pw(xn[0], [(p['level0'][0], 'lrelu'), (p['level0'][1], 'lrelu'),
                         (p['mlps'][0][0], 'lrelu'), (p['mlps'][0][1], 'lrelu')])
             for (p, xz, xn) in encs]

    specs = []
    for (p, xz, xn), f in zip(encs, f_lvl):
        idx = _knn(xz[0], xn[1], 16)
        table = _pad128(jnp.concatenate(
            [xn[0], f], axis=-1).reshape(B * xz[0].shape[2], -1))
        specs.append((table, _flat_idx(idx, xz[0].shape[2])))
    gs = _sc_gather_multi(specs)

    feats1 = []
    for (p, xz, xn), g in zip(encs, gs):
        Nq = xz[1].shape[2]
        gg = g.reshape(16, B, Nq, -1)
        feats1.append(_point_conv_post(p['convs'][0], gg, xn[1], 96))

    f_lvl2 = [_pw(f, [(e[0]['mlps'][1][0], 'lrelu'),
                      (e[0]['mlps'][1][1], 'lrelu')])
              for e, f in zip(encs, feats1)]

    xyz1 = xyzs1[2]
    xyz1n = xyzs1n[2]
    specs = []
    for (p, xz, xn), f in zip(encs, f_lvl2):
        idx = _knn(xz[1], xn[2], 16)
        table = _pad128(jnp.concatenate(
            [xn[1], f], axis=-1).reshape(B * xz[1].shape[2], -1))
        specs.append((table, _flat_idx(idx, xz[1].shape[2])))
    knn_idx = _knn(xyz1, xyz1n, 32)
    xyz1_rows = _pad128(xyz1n.reshape(B * 2048, 3))
    specs.append((xyz1_rows, _flat_idx(knn_idx, 2048)))
    gs = _sc_gather_multi(specs)

    feats2 = []
    for (p, xz, xn), g in zip(encs, gs[:3]):
        gg = g.reshape(16, B, 2048, -1)
        feats2.append(_point_conv_post(p['convs'][1], gg, xn[2], 128))
    feat1, feat2, featc = feats2
    featc = _conv1d_n(params['cnet_aligner'], featc)

    rel32 = gs[3].reshape(32, B, 2048, -1)[..., :3] - xyz1n[None]

    mp = params['motion_encoder']
    gp = params['gru']
    fp = params['flow_head']
    cp = params['correlation']

    wnets = [mp['conv_c1'], mp['conv_f1'], mp['conv_f2'], mp['conv'],
             gp['conv_z'], gp['conv_r'], gp['conv_q'],
             fp['conv1'], fp['conv2']]
    wcat = {'w': jnp.concatenate([q['wnet']['w'] for q in wnets], axis=0),
            'b': jnp.concatenate([q['wnet']['b'] for q in wnets], axis=0)}
    wall = _pw(rel32.reshape(32 * B, 2048, 3), [(wcat, 'none')])
    wall = wall.reshape(32, B, 2048, -1)
    wf = {}
    off = 0
    for name, q in zip(['c1', 'f1', 'f2', 'mc', 'z', 'r', 'q', 'fh1', 'fh2'],
                       wnets):
        C = q['wnet']['w'].shape[0]
        wf[name] = wall[..., off:off + C]
        off += C

    cvT = [_bmm(feat2, jnp.transpose(feat1, (0, 2, 1))) / 128.0]
    Ns2 = [2048, 1024, 512, 256]
    for i in range(1, 4):
        idx = _knn(xyzs2[2 + i - 1], xyzs2n[2 + i], 3)
        table = cvT[i - 1].reshape(B * Ns2[i - 1], 2048)
        g = _sc_gather_multi([(table, _flat_idx(idx, Ns2[i - 1]))])[0]
        cvT.append(jnp.mean(g.reshape(3, B, Ns2[i], 2048), axis=0))
    cv_flat = [c.reshape(B * Ns2[i] * 16, 128) for i, c in enumerate(cvT)]

    h = jnp.tanh(featc[..., :128])
    x = jax.nn.relu(featc[..., 128:])

    xyzs2c = xyzs2[2:]
    xyzs2cn = xyzs2n[2:]
    lane_eye = jnp.eye(128, dtype=jnp.float32)
    n_idx = jnp.arange(2048, dtype=jnp.int32)

    flow_preds = []
    flow_pred = jnp.zeros_like(xyz1n)
    for it in range(N_ITERS):
        if it > 0:
            table = _pad128(jnp.concatenate(
                [xyz1n, flow_pred], axis=-1).reshape(B * 2048, 6))
            specs = []
            for ln in xyzs2cn:
                idx = _knn(xyz1, ln, 3)
                specs.append((table, _flat_idx(idx, 2048)))
            gs = _sc_gather_multi(specs)
            xyzs2_warp = []
            xyzs2_warp_n = []
            for ln, g in zip(xyzs2cn, gs):
                Nl = ln.shape[1]
                wn = ln - _interp_post(g.reshape(3, B, Nl, -1), ln)
                xyzs2_warp_n.append(wn)
                xyzs2_warp.append(jnp.transpose(wn, (0, 2, 1)))
        else:
            xyzs2_warp = xyzs2c
            xyzs2_warp_n = xyzs2cn

        specs = []
        idxs = []
        for i, xw in enumerate(xyzs2_warp):
            Ni = xw.shape[2]
            idx = _knn(xw, xyz1n, 16)
            idxs.append(idx)
            specs.append((_pad128(xyzs2_warp_n[i].reshape(B * Ni, 3)),
                          _flat_idx(idx, Ni)))
        for i in range(4):
            Ni = xyzs2_warp[i].shape[2]
            boff = (jnp.arange(B, dtype=jnp.int32) * Ni)[:, None, None]
            frow = (idxs[i] + boff) * 16 + (n_idx[None, :, None] // 128)
            specs.append((cv_flat[i],
                          jnp.transpose(frow, (2, 0, 1)).reshape(-1)))
        gs = _sc_gather_multi(specs)

        feats = []
        for i in range(4):
            kxyz = gs[i].reshape(16, B, 2048, -1)[..., :3]
            rel = kxyz - xyz1n[None]
            gflat = gs[4 + i].reshape(16, B, 16, 128, 128)
            corr = jnp.sum(gflat * lane_eye, axis=-1).reshape(16, B, 2048)
            feats.append(jnp.concatenate([rel, corr[..., None]], axis=-1))
        feats = jnp.stack(feats, axis=0)
        hc = feats
        for lp in cp['cost_mlp']:
            hc = jax.nn.relu(jnp.einsum('ljbnc,oc->ljbno', hc, lp['w'])
                             + lp['b'])
        hc = jnp.sum(hc, axis=1)
        costs = jnp.concatenate([hc[i] for i in range(4)], axis=-1)
        corr = _lrelu(_conv1d_n(cp['merge'], costs))

        lin_c1 = _conv1d_n(mp['conv_c1']['lin'], corr)
        lin_f1 = _conv1d_n(mp['conv_f1']['lin'], flow_pred)
        gs = _sc_gather_multi([
            (lin_c1.reshape(B * 2048, 128),
             _flat_idx(knn_idx[:, :, :16], 2048)),
            (_pad128(lin_f1.reshape(B * 2048, 32)), _flat_idx(knn_idx, 2048)),
        ])
        corr_feat = _dw_post(gs[0].reshape(16, B, 2048, -1), wf['c1'][:16], 128)
        flow_feat = _dw_post(gs[1].reshape(32, B, 2048, -1), wf['f1'], 32)
        lin_f2 = _conv1d_n(mp['conv_f2']['lin'], flow_feat)
        g = _sc_gather_multi([(_pad128(lin_f2.reshape(B * 2048, 16)),
                               _flat_idx(knn_idx[:, :, :16], 2048))])[0]
        flow_feat = _dw_post(g.reshape(16, B, 2048, -1), wf['f2'][:16], 16)
        lin_mc = _conv1d_n(mp['conv']['lin'],
                           jnp.concatenate([corr_feat, flow_feat], axis=-1))
        g = _sc_gather_multi([(_pad128(lin_mc.reshape(B * 2048, 125)),
                               _flat_idx(knn_idx[:, :, :16], 2048))])[0]
        mfeat = _dw_post(g.reshape(16, B, 2048, -1), wf['mc'][:16], 125)
        motion_feat = jnp.concatenate([mfeat, flow_pred], axis=-1)

        xmf = jnp.concatenate([x, motion_feat], axis=-1)
        hx = jnp.concatenate([h, xmf], axis=-1)
        lin_z = _conv1d_n(gp['conv_z']['lin'], hx)
        lin_r = _conv1d_n(gp['conv_r']['lin'], hx)
        gs = _sc_gather_multi([
            (lin_z.reshape(B * 2048, 128), _flat_idx(knn_idx[:, :, :4], 2048)),
            (lin_r.reshape(B * 2048, 128), _flat_idx(knn_idx[:, :, :4], 2048)),
        ])
        z = jax.nn.sigmoid(_dw_post(gs[0].reshape(4, B, 2048, -1),
                                    wf['z'][:4], 128, act=False))
        r = jax.nn.sigmoid(_dw_post(gs[1].reshape(4, B, 2048, -1),
                                    wf['r'][:4], 128, act=False))
        lin_q = _conv1d_n(gp['conv_q']['lin'],
                          jnp.concatenate([r * h, xmf], axis=-1))
        g = _sc_gather_multi([(lin_q.reshape(B * 2048, 128),
                               _flat_idx(knn_idx[:, :, :4], 2048))])[0]
        q = jnp.tanh(_dw_post(g.reshape(4, B, 2048, -1),
                              wf['q'][:4], 128, act=False))
        h = (1 - z) * h + z * q

        lin1 = _conv1d_n(fp['conv1']['lin'], h)
        g = _sc_gather_multi([(lin1.reshape(B * 2048, 128),
                               _flat_idx(knn_idx, 2048))])[0]
        f = _dw_post(g.reshape(32, B, 2048, -1), wf['fh1'], 128)
        lin2 = _conv1d_n(fp['conv2']['lin'], f)
        g = _sc_gather_multi([(_pad128(lin2.reshape(B * 2048, 64)),
                               _flat_idx(knn_idx, 2048))])[0]
        f = _dw_post(g.reshape(32, B, 2048, -1), wf['fh2'], 64)
        flow_delta = _conv1d_n(fp['fc'], f)
        flow_pred = flow_pred + flow_delta
        flow_preds.append(flow_pred)

    idx_up = _knn(xyz1, xyzs1n[0], 3)
    fidx = _flat_idx(idx_up, 2048)
    specs = [(_pad128(jnp.concatenate([xyz1n, fpred], axis=-1)
                      .reshape(B * 2048, 6)), fidx)
             for fpred in flow_preds]
    gs = _sc_gather_multi(specs)
    ups = [jnp.transpose(_interp_post(g.reshape(3, B, 8192, -1), xyzs1n[0]),
                         (0, 2, 1)) for g in gs]
    return jnp.stack(ups, axis=0)
